# baseline (speedup 1.0000x reference)
import jax
import jax.numpy as jnp
from jax.experimental import pallas as pl

U = 100000
I = 100000
N = U + I
D = 64
MOM = 0.05


def _pred_body(x_ref, w_ref, b_ref, o_ref):
    o_ref[...] = jnp.dot(x_ref[...], w_ref[...],
                         preferred_element_type=jnp.float32) + b_ref[...]


def kernel(user_emb, item_emb, W, b, adj_vals, u_his, i_his, adj_row, adj_col, user, item):
    ego = jnp.concatenate([user_emb, item_emb], axis=0)
    acc = ego
    for _ in range(2):
        ego = jax.ops.segment_sum(adj_vals[:, None] * ego[adj_col], adj_row,
                                  num_segments=N)
        acc = acc + ego
    mean = acc / 3.0
    u_online = mean[:U][user]
    i_online = mean[U:][item]
    u_target = u_his[user] * MOM + u_online * (1.0 - MOM)
    i_target = i_his[item] * MOM + i_online * (1.0 - MOM)
    x = jnp.concatenate([u_online, i_online], axis=0)
    Bt = x.shape[0]
    preds = pl.pallas_call(
        _pred_body,
        grid=(Bt // 2048,),
        in_specs=[
            pl.BlockSpec((2048, D), lambda i: (i, 0)),
            pl.BlockSpec((D, D), lambda i: (0, 0)),
            pl.BlockSpec((1, D), lambda i: (0, 0)),
        ],
        out_specs=pl.BlockSpec((2048, D), lambda i: (i, 0)),
        out_shape=jax.ShapeDtypeStruct((Bt, D), jnp.float32),
    )(x, W.T, b[None, :])
    u_pred = preds[:Bt // 2]
    i_pred = preds[Bt // 2:]
    return (u_pred, u_target, i_pred, i_target)


# speedup vs baseline: 3.1882x; 3.1882x over previous
import functools

import jax
import jax.numpy as jnp
from jax import lax
from jax.experimental import pallas as pl
from jax.experimental.pallas import tpu as pltpu
from jax.experimental.pallas import tpu_sc as plsc

U = 100000
N = 200000
D = 64
NNZ = 625000
MOM = 0.05
G = 128
NC = 2
NS = 16
NPASS = 4
DS = D // NPASS
NBLK = 306
EH = NBLK * NS * G
RPT = 6256


def _splat(vv, j):
    idx = jnp.full((16, 1), j, jnp.int32)
    dn = lax.GatherDimensionNumbers(
        offset_dims=(), collapsed_slice_dims=(0,), start_index_map=(0,))
    return lax.gather(vv, idx, dn, (1,),
                      mode=lax.GatherScatterMode.PROMISE_IN_BOUNDS)


def _mesh():
    return plsc.VectorSubcoreMesh(core_axis_name="c", subcore_axis_name="s")


def _propagate(srcs, bcol, bval, blidx, zeros):
    oshape = jax.ShapeDtypeStruct((N, DS), jnp.float32)

    @functools.partial(
        pl.kernel,
        mesh=_mesh(),
        compiler_params=pltpu.CompilerParams(use_tc_tiling_on_sc=False),
        out_type=(oshape,) * NPASS,
        scratch_types=[
            pltpu.VMEM((G,), jnp.int32),
            pltpu.VMEM((G,), jnp.float32),
            pltpu.VMEM((G,), jnp.int32),
            pltpu.VMEM((G, DS), jnp.float32),
            pltpu.VMEM_SHARED((U, DS), jnp.float32),
            pltpu.SemaphoreType.DMA,
        ],
    )
    def k(s0, s1, s2, s3, bcol_h, bval_h, blidx_h, zeros_h,
          o0, o1, o2, o3, col_v, val_v, lidx_v, rows_v, acc_sh, sem):
        cid = lax.axis_index("c")
        sid = lax.axis_index("s")
        iota = lax.iota(jnp.int32, 16)
        eoff = pl.multiple_of(cid * EH, EH)
        acc_off = pl.multiple_of(
            jnp.minimum(sid * RPT, U - RPT).astype(jnp.int32), 8)
        wb_off = pl.multiple_of(cid * U + acc_off, 8)

        for src_h, out_h in zip((s0, s1, s2, s3), (o0, o1, o2, o3)):
            pltpu.sync_copy(zeros_h, acc_sh.at[pl.ds(acc_off, RPT)])
            plsc.subcore_barrier()

            def blk_body(i, bc, src_h=src_h):
                start = pl.multiple_of(eoff + (i * NS + sid) * G, G)
                pltpu.sync_copy(bcol_h.at[pl.ds(start, G)], col_v)
                pltpu.sync_copy(bval_h.at[pl.ds(start, G)], val_v)
                pltpu.sync_copy(blidx_h.at[pl.ds(start, G)], lidx_v)
                pltpu.async_copy(src_h.at[col_v], rows_v, sem).wait()

                def grp_body(g, gc):
                    g16 = pl.multiple_of(g * 16, 16)
                    vv = val_v[pl.ds(g16, 16)]
                    for j in range(16):
                        spl = _splat(vv, j)
                        rows_v[g16 + j, :] = rows_v[g16 + j, :] * spl
                    return gc

                lax.fori_loop(0, G // 16, grp_body, 0)
                pltpu.sync_copy(rows_v, acc_sh.at[lidx_v], add=True)
                return bc

            lax.fori_loop(0, NBLK, blk_body, 0)
            plsc.subcore_barrier()
            pltpu.sync_copy(acc_sh.at[pl.ds(acc_off, RPT)],
                            out_h.at[pl.ds(wb_off, RPT)])
            plsc.subcore_barrier()

    return k(srcs[0], srcs[1], srcs[2], srcs[3], bcol, bval, blidx, zeros)


def _batch_gather(ego0c, ego1, ego2, hisc, nodes):
    TB = nodes.shape[0]
    PW = TB // (NC * NS)
    NB = PW // G
    oshape = jax.ShapeDtypeStruct((TB, D), jnp.float32)

    @functools.partial(
        pl.kernel,
        mesh=_mesh(),
        compiler_params=pltpu.CompilerParams(use_tc_tiling_on_sc=False),
        out_type=(oshape, oshape, oshape, oshape),
        scratch_types=[
            pltpu.VMEM((G,), jnp.int32),
            pltpu.VMEM((G, D), jnp.float32),
            pltpu.SemaphoreType.DMA,
        ],
    )
    def k(t0, t1, t2, t3, nodes_h, g0, g1, g2, g3, idx_v, rows_v, sem):
        cid = lax.axis_index("c")
        sid = lax.axis_index("s")
        wid = sid * NC + cid
        base = wid * PW

        def blk_body(i, bc):
            p0 = pl.multiple_of(base + i * G, G)
            pltpu.sync_copy(nodes_h.at[pl.ds(p0, G)], idx_v)
            for t_h, g_h in ((t0, g0), (t1, g1), (t2, g2), (t3, g3)):
                pltpu.async_copy(t_h.at[idx_v], rows_v, sem).wait()
                pltpu.sync_copy(rows_v, g_h.at[pl.ds(p0, G)])
            return bc

        lax.fori_loop(0, NB, blk_body, 0)

    return k(ego0c, ego1, ego2, hisc, nodes)


def _tc_tail_body(ge_ref, g1_ref, g2_ref, gh_ref, w_ref, b_ref,
                  pred_ref, targ_ref):
    online = (ge_ref[...] + g1_ref[...] + g2_ref[...]) * (1.0 / 3.0)
    targ_ref[...] = gh_ref[...] * MOM + online * (1.0 - MOM)
    pred_ref[...] = jnp.dot(online, w_ref[...],
                            preferred_element_type=jnp.float32) + b_ref[...]


def _tc_tail(g_emb, g_l1, g_l2, g_his, Wt, b2d):
    TB = g_emb.shape[0]
    blk = 2048
    return pl.pallas_call(
        _tc_tail_body,
        grid=(TB // blk,),
        in_specs=[
            pl.BlockSpec((blk, D), lambda i: (i, 0)),
            pl.BlockSpec((blk, D), lambda i: (i, 0)),
            pl.BlockSpec((blk, D), lambda i: (i, 0)),
            pl.BlockSpec((blk, D), lambda i: (i, 0)),
            pl.BlockSpec((D, D), lambda i: (0, 0)),
            pl.BlockSpec((1, D), lambda i: (0, 0)),
        ],
        out_specs=[
            pl.BlockSpec((blk, D), lambda i: (i, 0)),
            pl.BlockSpec((blk, D), lambda i: (i, 0)),
        ],
        out_shape=[
            jax.ShapeDtypeStruct((TB, D), jnp.float32),
            jax.ShapeDtypeStruct((TB, D), jnp.float32),
        ],
    )(g_emb, g_l1, g_l2, g_his, Wt, b2d)


def _pad_half(x, fill):
    pad = jnp.full((EH - NNZ,), fill, x.dtype)
    return jnp.concatenate([x[:NNZ], pad, x[NNZ:], pad])


def _slices(x):
    return tuple(x[:, p * DS:(p + 1) * DS] for p in range(NPASS))


def kernel(user_emb, item_emb, W, b, adj_vals, u_his, i_his, adj_row, adj_col,
           user, item):
    B = user.shape[0]
    bcol = _pad_half(adj_col.astype(jnp.int32), 0)
    bval = _pad_half(adj_vals, 0.0)
    blidx = _pad_half((adj_row % U).astype(jnp.int32), 0)
    zeros = jnp.zeros((RPT, DS), jnp.float32)

    ego0 = jnp.concatenate([user_emb, item_emb], axis=0)
    l1 = _propagate(_slices(ego0), bcol, bval, blidx, zeros)
    l2 = _propagate(l1, bcol, bval, blidx, zeros)
    ego1 = jnp.concatenate(l1, axis=1)
    ego2 = jnp.concatenate(l2, axis=1)

    nodes = jnp.concatenate(
        [user.astype(jnp.int32), item.astype(jnp.int32) + U])
    hisc = jnp.concatenate([u_his, i_his], axis=0)
    g_emb, g_l1, g_l2, g_his = _batch_gather(ego0, ego1, ego2, hisc, nodes)

    pred, targ = _tc_tail(g_emb, g_l1, g_l2, g_his, W.T, b[None, :])
    return (pred[:B], targ[:B], pred[B:], targ[B:])


# speedup vs baseline: 6.7843x; 2.1279x over previous
import functools

import jax
import jax.numpy as jnp
from jax import lax
from jax.experimental import pallas as pl
from jax.experimental.pallas import tpu as pltpu
from jax.experimental.pallas import tpu_sc as plsc

U = 100000
N = 200000
D = 64
NNZ = 625000
MOM = 0.05
G = 128
NC = 2
NS = 16
NPASS = 4
DS = D // NPASS
NBLK = 306
EH = NBLK * NS * G
RPT = 6256


def _splat(vv, j):
    idx = jnp.full((16, 1), j, jnp.int32)
    dn = lax.GatherDimensionNumbers(
        offset_dims=(), collapsed_slice_dims=(0,), start_index_map=(0,))
    return lax.gather(vv, idx, dn, (1,),
                      mode=lax.GatherScatterMode.PROMISE_IN_BOUNDS)


def _mesh():
    return plsc.VectorSubcoreMesh(core_axis_name="c", subcore_axis_name="s")


def _propagate(srcs, erec, bval, zeros):
    oshape = jax.ShapeDtypeStruct((N, DS), jnp.float32)

    @functools.partial(
        pl.kernel,
        mesh=_mesh(),
        compiler_params=pltpu.CompilerParams(use_tc_tiling_on_sc=False),
        out_type=(oshape,) * NPASS,
        scratch_types=[
            pltpu.VMEM((3, 2, G), jnp.int32),
            pltpu.VMEM((3, G), jnp.float32),
            pltpu.VMEM((3, G, DS), jnp.float32),
            pltpu.VMEM_SHARED((U, DS), jnp.float32),
            pltpu.SemaphoreType.DMA,
            pltpu.SemaphoreType.DMA,
            pltpu.SemaphoreType.DMA,
            pltpu.SemaphoreType.DMA,
            pltpu.SemaphoreType.DMA,
            pltpu.SemaphoreType.DMA,
        ],
    )
    def k(s0, s1, s2, s3, erec_h, bval_h, zeros_h,
          o0, o1, o2, o3, er_v, val_v, rows_v, acc_sh,
          se0, se1, se2, sg0, sg1, sg2):
        sem_e = (se0, se1, se2)
        sem_g = (sg0, sg1, sg2)
        cid = lax.axis_index("c")
        sid = lax.axis_index("s")
        acc_off = pl.multiple_of(
            jnp.minimum(sid * RPT, U - RPT).astype(jnp.int32), 8)
        wb_off = pl.multiple_of(cid * U + acc_off, 8)
        half_rows = EH // G

        for src_h, out_h in zip((s0, s1, s2, s3), (o0, o1, o2, o3)):
            pltpu.sync_copy(zeros_h, acc_sh.at[pl.ds(acc_off, RPT)])
            plsc.subcore_barrier()

            def body(i3, bc, src_h=src_h):
                rows = []
                for b in range(3):
                    row = cid * half_rows + ((i3 * 3 + b) * NS + sid)
                    rows.append(row)
                    pltpu.async_copy(erec_h.at[row], er_v.at[b], sem_e[b])
                    start = pl.multiple_of(row * G, G)
                    pltpu.async_copy(
                        bval_h.at[pl.ds(start, G)], val_v.at[b], sem_e[b])
                gat = []
                for b in range(3):
                    pltpu.make_async_copy(
                        erec_h.at[rows[b]], er_v.at[b], sem_e[b]).wait()
                    start = pl.multiple_of(rows[b] * G, G)
                    pltpu.make_async_copy(
                        bval_h.at[pl.ds(start, G)], val_v.at[b],
                        sem_e[b]).wait()
                    gat.append(pltpu.async_copy(
                        src_h.at[er_v.at[b, 0]], rows_v.at[b], sem_g[b]))
                for b in range(3):
                    gat[b].wait()

                    def grp_body(g, gc, b=b):
                        g16 = pl.multiple_of(g * 16, 16)
                        vv = val_v[b, pl.ds(g16, 16)]
                        for j in range(16):
                            spl = _splat(vv, j)
                            rows_v[b, g16 + j, :] = (
                                rows_v[b, g16 + j, :] * spl)
                        return gc

                    lax.fori_loop(0, G // 16, grp_body, 0)
                    pltpu.sync_copy(rows_v.at[b],
                                    acc_sh.at[er_v.at[b, 1]], add=True)
                return bc

            lax.fori_loop(0, NBLK // 3, body, 0)
            plsc.subcore_barrier()
            pltpu.sync_copy(acc_sh.at[pl.ds(acc_off, RPT)],
                            out_h.at[pl.ds(wb_off, RPT)])
            plsc.subcore_barrier()

    return k(srcs[0], srcs[1], srcs[2], srcs[3], erec, bval, zeros)


def _batch_gather(ego0c, ego1, ego2, hisc, nodes):
    TB = nodes.shape[0]
    PW = TB // (NC * NS)
    NB = PW // G
    oshape = jax.ShapeDtypeStruct((TB, D), jnp.float32)

    @functools.partial(
        pl.kernel,
        mesh=_mesh(),
        compiler_params=pltpu.CompilerParams(use_tc_tiling_on_sc=False),
        out_type=(oshape, oshape, oshape, oshape),
        scratch_types=[
            pltpu.VMEM((G,), jnp.int32),
            pltpu.VMEM((G, D), jnp.float32),
            pltpu.SemaphoreType.DMA,
        ],
    )
    def k(t0, t1, t2, t3, nodes_h, g0, g1, g2, g3, idx_v, rows_v, sem):
        cid = lax.axis_index("c")
        sid = lax.axis_index("s")
        wid = sid * NC + cid
        base = wid * PW

        def blk_body(i, bc):
            p0 = pl.multiple_of(base + i * G, G)
            pltpu.sync_copy(nodes_h.at[pl.ds(p0, G)], idx_v)
            for t_h, g_h in ((t0, g0), (t1, g1), (t2, g2), (t3, g3)):
                pltpu.async_copy(t_h.at[idx_v], rows_v, sem).wait()
                pltpu.sync_copy(rows_v, g_h.at[pl.ds(p0, G)])
            return bc

        lax.fori_loop(0, NB, blk_body, 0)

    return k(ego0c, ego1, ego2, hisc, nodes)


def _tc_tail_body(ge_ref, g1_ref, g2_ref, gh_ref, w_ref, b_ref,
                  pred_ref, targ_ref):
    online = (ge_ref[...] + g1_ref[...] + g2_ref[...]) * (1.0 / 3.0)
    targ_ref[...] = gh_ref[...] * MOM + online * (1.0 - MOM)
    pred_ref[...] = jnp.dot(online, w_ref[...],
                            preferred_element_type=jnp.float32) + b_ref[...]


def _tc_tail(g_emb, g_l1, g_l2, g_his, Wt, b2d):
    TB = g_emb.shape[0]
    blk = 2048
    return pl.pallas_call(
        _tc_tail_body,
        grid=(TB // blk,),
        in_specs=[
            pl.BlockSpec((blk, D), lambda i: (i, 0)),
            pl.BlockSpec((blk, D), lambda i: (i, 0)),
            pl.BlockSpec((blk, D), lambda i: (i, 0)),
            pl.BlockSpec((blk, D), lambda i: (i, 0)),
            pl.BlockSpec((D, D), lambda i: (0, 0)),
            pl.BlockSpec((1, D), lambda i: (0, 0)),
        ],
        out_specs=[
            pl.BlockSpec((blk, D), lambda i: (i, 0)),
            pl.BlockSpec((blk, D), lambda i: (i, 0)),
        ],
        out_shape=[
            jax.ShapeDtypeStruct((TB, D), jnp.float32),
            jax.ShapeDtypeStruct((TB, D), jnp.float32),
        ],
    )(g_emb, g_l1, g_l2, g_his, Wt, b2d)


def _pad_half(x, fill):
    pad = jnp.full((EH - NNZ,), fill, x.dtype)
    return jnp.concatenate([x[:NNZ], pad, x[NNZ:], pad])


def _slices(x):
    return tuple(x[:, p * DS:(p + 1) * DS] for p in range(NPASS))


def kernel(user_emb, item_emb, W, b, adj_vals, u_his, i_his, adj_row, adj_col,
           user, item):
    B = user.shape[0]
    colp = _pad_half(adj_col.astype(jnp.int32), 0)
    valp = _pad_half(adj_vals, 0.0)
    lidxp = _pad_half((adj_row % U).astype(jnp.int32), 0)
    erec = jnp.stack([colp.reshape(-1, G), lidxp.reshape(-1, G)], axis=1)
    zeros = jnp.zeros((RPT, DS), jnp.float32)

    ego0 = jnp.concatenate([user_emb, item_emb], axis=0)
    l1 = _propagate(_slices(ego0), erec, valp, zeros)
    l2 = _propagate(l1, erec, valp, zeros)
    ego1 = jnp.concatenate(l1, axis=1)
    ego2 = jnp.concatenate(l2, axis=1)

    nodes = jnp.concatenate(
        [user.astype(jnp.int32), item.astype(jnp.int32) + U])
    hisc = jnp.concatenate([u_his, i_his], axis=0)
    g_emb, g_l1, g_l2, g_his = _batch_gather(ego0, ego1, ego2, hisc, nodes)

    pred, targ = _tc_tail(g_emb, g_l1, g_l2, g_his, W.T, b[None, :])
    return (pred[:B], targ[:B], pred[B:], targ[B:])


# speedup vs baseline: 7.1468x; 1.0534x over previous
import functools

import jax
import jax.numpy as jnp
from jax import lax
from jax.experimental import pallas as pl
from jax.experimental.pallas import tpu as pltpu
from jax.experimental.pallas import tpu_sc as plsc

U = 100000
N = 200000
D = 64
NNZ = 625000
MOM = 0.05
G = 128
NC = 2
NS = 16
NPASS = 4
DS = D // NPASS
NBLK = 306
EH = NBLK * NS * G
RPT = 6256


def _splat(vv, j):
    idx = jnp.full((16, 1), j, jnp.int32)
    dn = lax.GatherDimensionNumbers(
        offset_dims=(), collapsed_slice_dims=(0,), start_index_map=(0,))
    return lax.gather(vv, idx, dn, (1,),
                      mode=lax.GatherScatterMode.PROMISE_IN_BOUNDS)


def _mesh():
    return plsc.VectorSubcoreMesh(core_axis_name="c", subcore_axis_name="s")


def _propagate(srcs, erec, bval, zeros):
    oshape = jax.ShapeDtypeStruct((N, DS), jnp.float32)

    @functools.partial(
        pl.kernel,
        mesh=_mesh(),
        compiler_params=pltpu.CompilerParams(use_tc_tiling_on_sc=False),
        out_type=(oshape,) * NPASS,
        scratch_types=[
            pltpu.VMEM((3, 2, G), jnp.int32),
            pltpu.VMEM((3, G), jnp.float32),
            pltpu.VMEM((3, G, DS), jnp.float32),
            pltpu.VMEM_SHARED((U, DS), jnp.float32),
            pltpu.SemaphoreType.DMA,
            pltpu.SemaphoreType.DMA,
            pltpu.SemaphoreType.DMA,
            pltpu.SemaphoreType.DMA,
            pltpu.SemaphoreType.DMA,
            pltpu.SemaphoreType.DMA,
            pltpu.SemaphoreType.DMA,
            pltpu.SemaphoreType.DMA,
            pltpu.SemaphoreType.DMA,
        ],
    )
    def k(s0, s1, s2, s3, erec_h, bval_h, zeros_h,
          o0, o1, o2, o3, er_v, val_v, rows_v, acc_sh,
          se0, se1, se2, sg0, sg1, sg2, ss0, ss1, ss2):
        sem_e = (se0, se1, se2)
        sem_g = (sg0, sg1, sg2)
        sem_s = (ss0, ss1, ss2)
        cid = lax.axis_index("c")
        sid = lax.axis_index("s")
        acc_off = pl.multiple_of(
            jnp.minimum(sid * RPT, U - RPT).astype(jnp.int32), 8)
        wb_off = pl.multiple_of(cid * U + acc_off, 8)
        half_rows = EH // G

        def _erow(i3, b):
            bl = jnp.minimum(i3 * 3 + b, NBLK - 1)
            return cid * half_rows + (bl * NS + sid)

        def _issue_edges(i3, b):
            row = _erow(i3, b)
            pltpu.async_copy(erec_h.at[row], er_v.at[b], sem_e[b])
            start = pl.multiple_of(row * G, G)
            pltpu.async_copy(
                bval_h.at[pl.ds(start, G)], val_v.at[b], sem_e[b])

        def _wait_edges(i3, b):
            row = _erow(i3, b)
            pltpu.make_async_copy(
                erec_h.at[row], er_v.at[b], sem_e[b]).wait()
            start = pl.multiple_of(row * G, G)
            pltpu.make_async_copy(
                bval_h.at[pl.ds(start, G)], val_v.at[b], sem_e[b]).wait()

        for src_h, out_h in zip((s0, s1, s2, s3), (o0, o1, o2, o3)):
            pltpu.sync_copy(zeros_h, acc_sh.at[pl.ds(acc_off, RPT)])
            plsc.subcore_barrier()
            for b in range(3):
                _issue_edges(0, b)

            def body(i3, bc, src_h=src_h):
                gat = []
                for b in range(3):
                    _wait_edges(i3, b)
                    gat.append(pltpu.async_copy(
                        src_h.at[er_v.at[b, 0]], rows_v.at[b], sem_g[b]))
                sca = []
                for b in range(3):
                    gat[b].wait()

                    def grp_body(g, gc, b=b):
                        g16 = pl.multiple_of(g * 16, 16)
                        vv = val_v[b, pl.ds(g16, 16)]
                        for j in range(16):
                            spl = _splat(vv, j)
                            rows_v[b, g16 + j, :] = (
                                rows_v[b, g16 + j, :] * spl)
                        return gc

                    lax.fori_loop(0, G // 16, grp_body, 0)
                    sca.append(pltpu.async_copy(
                        rows_v.at[b], acc_sh.at[er_v.at[b, 1]], sem_s[b],
                        add=True))
                for b in range(3):
                    sca[b].wait()
                for b in range(3):
                    _issue_edges(i3 + 1, b)
                return bc

            lax.fori_loop(0, NBLK // 3, body, 0)
            for b in range(3):
                _wait_edges(NBLK // 3, b)
            plsc.subcore_barrier()
            pltpu.sync_copy(acc_sh.at[pl.ds(acc_off, RPT)],
                            out_h.at[pl.ds(wb_off, RPT)])
            plsc.subcore_barrier()

    return k(srcs[0], srcs[1], srcs[2], srcs[3], erec, bval, zeros)


def _batch_gather(ego0c, ego1, ego2, hisc, nodes):
    TB = nodes.shape[0]
    PW = TB // (NC * NS)
    NB = PW // G
    oshape = jax.ShapeDtypeStruct((TB, D), jnp.float32)

    @functools.partial(
        pl.kernel,
        mesh=_mesh(),
        compiler_params=pltpu.CompilerParams(use_tc_tiling_on_sc=False),
        out_type=(oshape, oshape, oshape, oshape),
        scratch_types=[
            pltpu.VMEM((G,), jnp.int32),
            pltpu.VMEM((G, D), jnp.float32),
            pltpu.SemaphoreType.DMA,
        ],
    )
    def k(t0, t1, t2, t3, nodes_h, g0, g1, g2, g3, idx_v, rows_v, sem):
        cid = lax.axis_index("c")
        sid = lax.axis_index("s")
        wid = sid * NC + cid
        base = wid * PW

        def blk_body(i, bc):
            p0 = pl.multiple_of(base + i * G, G)
            pltpu.sync_copy(nodes_h.at[pl.ds(p0, G)], idx_v)
            for t_h, g_h in ((t0, g0), (t1, g1), (t2, g2), (t3, g3)):
                pltpu.async_copy(t_h.at[idx_v], rows_v, sem).wait()
                pltpu.sync_copy(rows_v, g_h.at[pl.ds(p0, G)])
            return bc

        lax.fori_loop(0, NB, blk_body, 0)

    return k(ego0c, ego1, ego2, hisc, nodes)


def _tc_tail_body(ge_ref, g1_ref, g2_ref, gh_ref, w_ref, b_ref,
                  pred_ref, targ_ref):
    online = (ge_ref[...] + g1_ref[...] + g2_ref[...]) * (1.0 / 3.0)
    targ_ref[...] = gh_ref[...] * MOM + online * (1.0 - MOM)
    pred_ref[...] = jnp.dot(online, w_ref[...],
                            preferred_element_type=jnp.float32) + b_ref[...]


def _tc_tail(g_emb, g_l1, g_l2, g_his, Wt, b2d):
    TB = g_emb.shape[0]
    blk = 2048
    return pl.pallas_call(
        _tc_tail_body,
        grid=(TB // blk,),
        in_specs=[
            pl.BlockSpec((blk, D), lambda i: (i, 0)),
            pl.BlockSpec((blk, D), lambda i: (i, 0)),
            pl.BlockSpec((blk, D), lambda i: (i, 0)),
            pl.BlockSpec((blk, D), lambda i: (i, 0)),
            pl.BlockSpec((D, D), lambda i: (0, 0)),
            pl.BlockSpec((1, D), lambda i: (0, 0)),
        ],
        out_specs=[
            pl.BlockSpec((blk, D), lambda i: (i, 0)),
            pl.BlockSpec((blk, D), lambda i: (i, 0)),
        ],
        out_shape=[
            jax.ShapeDtypeStruct((TB, D), jnp.float32),
            jax.ShapeDtypeStruct((TB, D), jnp.float32),
        ],
    )(g_emb, g_l1, g_l2, g_his, Wt, b2d)


def _pad_half(x, fill):
    pad = jnp.full((EH - NNZ,), fill, x.dtype)
    return jnp.concatenate([x[:NNZ], pad, x[NNZ:], pad])


def _slices(x):
    return tuple(x[:, p * DS:(p + 1) * DS] for p in range(NPASS))


def kernel(user_emb, item_emb, W, b, adj_vals, u_his, i_his, adj_row, adj_col,
           user, item):
    B = user.shape[0]
    colp = _pad_half(adj_col.astype(jnp.int32), 0)
    valp = _pad_half(adj_vals, 0.0)
    lidxp = _pad_half((adj_row % U).astype(jnp.int32), 0)
    erec = jnp.stack([colp.reshape(-1, G), lidxp.reshape(-1, G)], axis=1)
    zeros = jnp.zeros((RPT, DS), jnp.float32)

    ego0 = jnp.concatenate([user_emb, item_emb], axis=0)
    l1 = _propagate(_slices(ego0), erec, valp, zeros)
    l2 = _propagate(l1, erec, valp, zeros)
    ego1 = jnp.concatenate(l1, axis=1)
    ego2 = jnp.concatenate(l2, axis=1)

    nodes = jnp.concatenate(
        [user.astype(jnp.int32), item.astype(jnp.int32) + U])
    hisc = jnp.concatenate([u_his, i_his], axis=0)
    g_emb, g_l1, g_l2, g_his = _batch_gather(ego0, ego1, ego2, hisc, nodes)

    pred, targ = _tc_tail(g_emb, g_l1, g_l2, g_his, W.T, b[None, :])
    return (pred[:B], targ[:B], pred[B:], targ[B:])
